# baseline (speedup 1.0000x reference)
import functools

import jax
import jax.numpy as jnp
from jax.experimental import pallas as pl

_HI = jax.lax.Precision.HIGHEST


def _dot(a, b):
    return jax.lax.dot_general(a.astype(jnp.bfloat16), b.astype(jnp.bfloat16),
                               (((1,), (0,)), ((), ())),
                               preferred_element_type=jnp.float32)


def _dot_exact(a, b):
    return jax.lax.dot_general(a, b, (((1,), (0,)), ((), ())),
                               precision=_HI,
                               preferred_element_type=jnp.float32)


def _vq_kernel(z_ref, cbt_ref, cb_ref, q_ref):
    z = z_ref[...]
    cbt = cbt_ref[...]
    cb = cb_ref[...]
    zn = jnp.sum(z * z, axis=1, keepdims=True)
    cbn = jnp.sum(cbt * cbt, axis=0, keepdims=True)
    d2 = (zn + cbn) - 2.0 * _dot(z, cbt)
    iota = jax.lax.broadcasted_iota(jnp.int32, d2.shape, 1)
    big = jnp.int32(2**30)
    v = jnp.min(d2, axis=1, keepdims=True)
    idx = jnp.min(jnp.where(d2 == v, iota, big), axis=1)
    oh = (iota == idx[:, None]).astype(jnp.float32)
    q_ref[...] = _dot_exact(oh, cb)


def _vq(z, codebook):
    M, C = z.shape
    K = codebook.shape[0]
    TM = 512
    return pl.pallas_call(
        _vq_kernel,
        grid=(M // TM,),
        in_specs=[pl.BlockSpec((TM, C), lambda i: (i, 0)),
                  pl.BlockSpec((C, K), lambda i: (0, 0)),
                  pl.BlockSpec((K, C), lambda i: (0, 0))],
        out_specs=pl.BlockSpec((TM, C), lambda i: (i, 0)),
        out_shape=jax.ShapeDtypeStruct((M, C), jnp.float32),
    )(z, codebook.T, codebook)


def _s2d(x, pad_cols):
    n, h, w, c = x.shape
    x = jnp.pad(x, ((0, 0), (1, 1), (1, 1), (0, 0)))
    hc, wc = (h + 2) // 2, (w + 2) // 2
    x = x.reshape(n, hc, 2, wc, 2, c).transpose(0, 1, 3, 2, 4, 5)
    x = x.reshape(n, hc, wc, 4 * c)
    return jnp.pad(x, ((0, 0), (0, 0), (0, pad_cols - wc), (0, 0)))


def _conv_s2_kernel(rows, cols, x_ref, w_ref, b_ref, o_ref):
    i = pl.program_id(1)
    acc = None
    for dr in range(2):
        for dc in range(2):
            a = x_ref[0, pl.ds(i * rows + dr, rows), dc:dc + cols, :]
            a = a.reshape(rows * cols, a.shape[-1])
            p = _dot(a, w_ref[2 * dr + dc])
            acc = p if acc is None else acc + p
    acc = jnp.maximum(acc + b_ref[...], 0.0)
    o_ref[0] = acc.reshape(rows, cols, -1)[:, :o_ref.shape[2], :]


def _conv_s2(x, w, b, out_hw, cols, rows):
    n, _, _, c = x.shape
    co = w.shape[0]
    xs = _s2d(x, cols + 8)
    wt = w.reshape(co, c, 2, 2, 2, 2).transpose(2, 4, 3, 5, 1, 0)
    wt = wt.reshape(4, 4 * c, co)
    k = 4 * c
    if k < 128:
        xs = jnp.pad(xs, ((0, 0), (0, 0), (0, 0), (0, 128 - k)))
        wt = jnp.pad(wt, ((0, 0), (0, 128 - k), (0, 0)))
        k = 128
    kern = functools.partial(_conv_s2_kernel, rows, cols)
    return pl.pallas_call(
        kern,
        grid=(n, out_hw // rows),
        in_specs=[pl.BlockSpec((1, out_hw + 1, cols + 8, k),
                               lambda b_, i: (b_, 0, 0, 0)),
                  pl.BlockSpec((4, k, co), lambda b_, i: (0, 0, 0)),
                  pl.BlockSpec((1, co), lambda b_, i: (0, 0))],
        out_specs=pl.BlockSpec((1, rows, out_hw, co),
                               lambda b_, i: (b_, i, 0, 0)),
        out_shape=jax.ShapeDtypeStruct((n, out_hw, out_hw, co), jnp.float32),
    )(xs, wt, b.reshape(1, co))


def _conv3_kernel(rows, cols, relu, x_ref, w_ref, b_ref, o_ref):
    i = pl.program_id(1)
    acc = None
    for dh in range(3):
        for dw in range(3):
            a = x_ref[0, pl.ds(i * rows + dh, rows), dw:dw + cols, :]
            a = a.reshape(rows * cols, a.shape[-1])
            p = _dot(a, w_ref[3 * dh + dw])
            acc = p if acc is None else acc + p
    acc = acc + b_ref[...]
    if relu:
        acc = jnp.maximum(acc, 0.0)
    o_ref[0] = acc.reshape(rows, cols, -1)[:, :o_ref.shape[2], :]


def _conv3(x, w, b, relu, rows):
    n, hw, _, c = x.shape
    co = w.shape[0]
    cols = hw + 8
    xp = jnp.pad(x, ((0, 0), (1, 1), (1, cols + 1 - hw), (0, 0)))
    wt = w.transpose(2, 3, 1, 0).reshape(9, c, co)
    kern = functools.partial(_conv3_kernel, rows, cols, relu)
    return pl.pallas_call(
        kern,
        grid=(n, hw // rows),
        in_specs=[pl.BlockSpec((1, hw + 2, cols + 2, c),
                               lambda b_, i: (b_, 0, 0, 0)),
                  pl.BlockSpec((9, c, co), lambda b_, i: (0, 0, 0)),
                  pl.BlockSpec((1, co), lambda b_, i: (0, 0))],
        out_specs=pl.BlockSpec((1, rows, hw, co), lambda b_, i: (b_, i, 0, 0)),
        out_shape=jax.ShapeDtypeStruct((n, hw, hw, co), jnp.float32),
    )(xp, wt, b.reshape(1, co))


def _deconv_kernel(rows, cols, relu, x_ref, w_ref, b_ref, o_ref):
    r = pl.program_id(1)
    i = pl.program_id(2)
    outs = []
    for s in range(2):
        acc = None
        for dr in range(2):
            for ds in range(2):
                a = x_ref[0, pl.ds(i * rows + dr + r, rows),
                          ds + s:ds + s + cols, :]
                a = a.reshape(rows * cols, a.shape[-1])
                p = _dot(a, w_ref[0, s, 2 * dr + ds])
                acc = p if acc is None else acc + p
        acc = acc + b_ref[...]
        if relu:
            acc = jnp.maximum(acc, 0.0)
        outs.append(acc.reshape(rows, cols, -1)[:, :o_ref.shape[3], :])
    o_ref[0, :, 0, :, :] = jnp.concatenate(outs, axis=-1)


def _deconv(x, w, b, relu, rows):
    n, hw, _, ci = x.shape
    co = w.shape[0]
    cols = hw + 8
    xp = jnp.pad(x, ((0, 0), (1, 1), (1, cols + 2 - hw), (0, 0)))
    wt = w.reshape(co, ci, 2, 2, 2, 2).transpose(3, 5, 2, 4, 1, 0)
    wt = wt.reshape(2, 2, 4, ci, co)
    kern = functools.partial(_deconv_kernel, rows, cols, relu)
    out = pl.pallas_call(
        kern,
        grid=(n, 2, hw // rows),
        in_specs=[pl.BlockSpec((1, hw + 2, cols + 3, ci),
                               lambda b_, r, i: (b_, 0, 0, 0)),
                  pl.BlockSpec((1, 2, 4, ci, co),
                               lambda b_, r, i: (r, 0, 0, 0, 0)),
                  pl.BlockSpec((1, co), lambda b_, r, i: (0, 0))],
        out_specs=pl.BlockSpec((1, rows, 1, hw, 2 * co),
                               lambda b_, r, i: (b_, i, r, 0, 0)),
        out_shape=jax.ShapeDtypeStruct((n, hw, 2, hw, 2 * co), jnp.float32),
    )(xp, wt, b.reshape(1, co))
    return out.reshape(n, 2 * hw, 2 * hw, co)


def kernel(data, enc_w1, enc_b1, enc_w2, enc_b2, enc_w3, enc_b3, codebook,
           dec_w1, dec_b1, dec_w2, dec_b2, dec_w3, dec_b3):
    x = data.transpose(0, 2, 3, 1)
    h = _conv_s2(x, enc_w1, enc_b1, 112, 120, 28)
    h = _conv_s2(h, enc_w2, enc_b2, 56, 64, 28)
    e_nhwc = _conv3(h, enc_w3, enc_b3, False, 28)
    B, H, W, C = e_nhwc.shape
    q_flat = _vq(e_nhwc.reshape(-1, C), codebook)
    q_nhwc = q_flat.reshape(B, H, W, C)
    h = _conv3(q_nhwc, dec_w1, dec_b1, True, 28)
    h = _deconv(h, dec_w2, dec_b2, True, 28)
    d_nhwc = _deconv(h, dec_w3, dec_b3, False, 28)
    d = d_nhwc.transpose(0, 3, 1, 2)
    e = e_nhwc.transpose(0, 3, 1, 2)
    q = q_nhwc.transpose(0, 3, 1, 2)
    return (d, e, q)
